# baseline (speedup 1.0000x reference)
import functools

import jax
import jax.numpy as jnp
from jax import lax
from jax.experimental import pallas as pl
from jax.experimental.pallas import tpu as pltpu
from jax.experimental.pallas import tpu_sc as plsc

NC = 2
NS = 16
NW = NC * NS
K = 128
DH = 64
GB = 4


def _build_sc_kernel(n, g, npad):
    rps = npad // NS
    rblk = rps // K
    assert rps % K == 0

    mesh = plsc.VectorSubcoreMesh(core_axis_name="c", subcore_axis_name="s")

    @functools.partial(
        pl.kernel,
        mesh=mesh,
        out_type=[
            jax.ShapeDtypeStruct((NC, 2, npad, DH), jnp.float32),
            jax.ShapeDtypeStruct((NC, npad, 16), jnp.float32),
        ],
        scratch_types=[
            pltpu.VMEM((g, K), jnp.int32),
            pltpu.VMEM((g, K), jnp.int32),
            pltpu.VMEM((GB, K, DH), jnp.float32),
            pltpu.VMEM((K, DH), jnp.float32),
            pltpu.VMEM((K, 16), jnp.float32),
            pltpu.VMEM((K, 16), jnp.float32),
            pltpu.VMEM_SHARED((npad, DH), jnp.float32),
            pltpu.VMEM_SHARED((npad, 16), jnp.float32),
            pltpu.SemaphoreType.DMA,
            pltpu.SemaphoreType.DMA,
        ],
        compiler_params=pltpu.CompilerParams(use_tc_tiling_on_sc=False),
    )
    def sc_agg(x0_hbm, x1_hbm, src_hbm, dst_hbm, agg_out, deg_out,
               src_v, dst_v, grp_v, rows_v, ones_v, deg_v, agg_sh, deg_sh,
               sem_g, sem_s):
        c = lax.axis_index("c")
        s = lax.axis_index("s")
        wid = s * NC + c
        base = s * rps

        def zero_rows_v():
            def zr(i, carry):
                def zc(j, carry2):
                    rows_v[i, pl.ds(j * 16, 16)] = jnp.zeros((16,), jnp.float32)
                    return carry2
                return lax.fori_loop(0, DH // 16, zc, carry)
            lax.fori_loop(0, K, zr, 0)

        def zero_own_agg_slice():
            for t in range(rblk):
                pltpu.sync_copy(rows_v, agg_sh.at[pl.ds(base + t * K, K)])

        zero_rows_v()

        def zd(i, carry):
            deg_v[i, :] = jnp.zeros((16,), jnp.float32)
            ones_v[i, :] = jnp.ones((16,), jnp.float32)
            return carry
        lax.fori_loop(0, K, zd, 0)

        zero_own_agg_slice()
        for t in range(rblk):
            pltpu.sync_copy(deg_v, deg_sh.at[pl.ds(base + t * K, K)])
        plsc.subcore_barrier()

        pltpu.sync_copy(src_hbm.at[wid], src_v)
        pltpu.sync_copy(dst_hbm.at[wid], dst_v)

        def body0(og, carry):
            g0 = og * GB
            gathers = [
                pltpu.async_copy(
                    x0_hbm.at[src_v.at[g0 + b]], grp_v.at[b], sem_g)
                for b in range(GB)
            ]
            for cp in gathers:
                cp.wait()
            scatters = []
            for b in range(GB):
                scatters.append(pltpu.async_copy(
                    grp_v.at[b], agg_sh.at[dst_v.at[g0 + b]], sem_s,
                    add=True))
                scatters.append(pltpu.async_copy(
                    ones_v, deg_sh.at[dst_v.at[g0 + b]], sem_s, add=True))
            for cp in scatters:
                cp.wait()
            return carry
        lax.fori_loop(0, g // GB, body0, 0)
        plsc.subcore_barrier()

        for t in range(rblk):
            sl = pl.ds(base + t * K, K)
            pltpu.sync_copy(agg_sh.at[sl], rows_v)
            pltpu.sync_copy(rows_v, agg_out.at[c, 0, sl])
            pltpu.sync_copy(deg_sh.at[sl], deg_v)
            pltpu.sync_copy(deg_v, deg_out.at[c, sl])
        zero_rows_v()
        zero_own_agg_slice()
        plsc.subcore_barrier()

        def body1(og, carry):
            g0 = og * GB
            gathers = [
                pltpu.async_copy(
                    x1_hbm.at[src_v.at[g0 + b]], grp_v.at[b], sem_g)
                for b in range(GB)
            ]
            for cp in gathers:
                cp.wait()
            scatters = [
                pltpu.async_copy(
                    grp_v.at[b], agg_sh.at[dst_v.at[g0 + b]], sem_s, add=True)
                for b in range(GB)
            ]
            for cp in scatters:
                cp.wait()
            return carry
        lax.fori_loop(0, g // GB, body1, 0)
        plsc.subcore_barrier()

        for t in range(rblk):
            sl = pl.ds(base + t * K, K)
            pltpu.sync_copy(agg_sh.at[sl], rows_v)
            pltpu.sync_copy(rows_v, agg_out.at[c, 1, sl])

    return sc_agg


def _tc_finish(agg_parts, deg_parts, x, wl_t, wr_t, b2, rblock):
    n, d = x.shape

    def body(agg_ref, deg_ref, x_ref, wl_ref, wr_ref, b_ref, o_ref):
        a0 = agg_ref[0, 0] + agg_ref[1, 0]
        a1 = agg_ref[0, 1] + agg_ref[1, 1]
        dg = jnp.maximum(deg_ref[0, :, 0:1] + deg_ref[1, :, 0:1], 1.0)
        m0 = a0 / dg
        m1 = a1 / dg
        acc = jnp.dot(m0, wl_ref[:DH, :], preferred_element_type=jnp.float32)
        acc = acc + jnp.dot(m1, wl_ref[DH:, :],
                            preferred_element_type=jnp.float32)
        acc = acc + jnp.dot(x_ref[...], wr_ref[...],
                            preferred_element_type=jnp.float32)
        o_ref[...] = acc + b_ref[...]

    return pl.pallas_call(
        body,
        grid=(n // rblock,),
        in_specs=[
            pl.BlockSpec((NC, 2, rblock, DH), lambda i: (0, 0, i, 0)),
            pl.BlockSpec((NC, rblock, 16), lambda i: (0, i, 0)),
            pl.BlockSpec((rblock, d), lambda i: (i, 0)),
            pl.BlockSpec((d, d), lambda i: (0, 0)),
            pl.BlockSpec((d, d), lambda i: (0, 0)),
            pl.BlockSpec((1, d), lambda i: (0, 0)),
        ],
        out_specs=pl.BlockSpec((rblock, d), lambda i: (i, 0)),
        out_shape=jax.ShapeDtypeStruct((n, d), jnp.float32),
    )(agg_parts, deg_parts, x, wl_t, wr_t, b2)


def kernel(x, edge_index, W_l, b_l, W_r):
    n, d = x.shape
    e = edge_index.shape[1]

    g = -(-e // (NW * K))
    g = -(-g // GB) * GB
    e_pad = NW * g * K
    npad = -(-(n + 1) // (NS * K)) * (NS * K)

    src = edge_index[0]
    dst = edge_index[1]
    pad = e_pad - e
    if pad:
        src = jnp.concatenate([src, jnp.zeros((pad,), jnp.int32)])
        dst = jnp.concatenate([dst, jnp.full((pad,), n, jnp.int32)])
    src3d = src.reshape(NW, g, K)
    dst3d = dst.reshape(NW, g, K)

    x0 = x[:, :DH]
    x1 = x[:, DH:]

    sc_agg = _build_sc_kernel(n, g, npad)
    agg_parts, deg_parts = sc_agg(x0, x1, src3d, dst3d)

    rblock = 400 if n % 400 == 0 else 8
    return _tc_finish(agg_parts, deg_parts, x, W_l.T, W_r.T,
                      b_l.reshape(1, d), rblock)


# speedup vs baseline: 1.7623x; 1.7623x over previous
import functools

import jax
import jax.numpy as jnp
from jax import lax
from jax.experimental import pallas as pl
from jax.experimental.pallas import tpu as pltpu
from jax.experimental.pallas import tpu_sc as plsc

NC = 2
NS = 16
NW = NC * NS
K = 128


def _slice_plan(rps):
    plan = []
    off = 0
    while off < rps:
        blk = min(K, rps - off)
        plan.append((off, blk))
        off += blk
    return plan


def _build_sc_kernel(n, d, g, npad):
    rps = npad // NS
    plan = _slice_plan(rps)

    mesh = plsc.VectorSubcoreMesh(core_axis_name="c", subcore_axis_name="s")

    @functools.partial(
        pl.kernel,
        mesh=mesh,
        out_type=[
            jax.ShapeDtypeStruct((NC, npad, d), jnp.float32),
            jax.ShapeDtypeStruct((NC, npad, 16), jnp.float32),
        ],
        scratch_types=[
            pltpu.VMEM((g, K), jnp.int32),
            pltpu.VMEM((g, K), jnp.int32),
            pltpu.VMEM((K, d), jnp.float32),
            pltpu.VMEM((K, 16), jnp.float32),
            pltpu.VMEM((K, 16), jnp.float32),
            pltpu.VMEM_SHARED((npad, d), jnp.float32),
            pltpu.VMEM_SHARED((npad, 16), jnp.float32),
            pltpu.SemaphoreType.DMA,
        ],
        compiler_params=pltpu.CompilerParams(use_tc_tiling_on_sc=False),
    )
    def sc_agg(x_hbm, src_hbm, dst_hbm, agg_out, deg_out,
               src_v, dst_v, rows_v, ones_v, deg_v, agg_sh, deg_sh, sem):
        c = lax.axis_index("c")
        s = lax.axis_index("s")
        wid = s * NC + c
        base = s * rps

        def zr(i, carry):
            def zc(j, carry2):
                rows_v[i, pl.ds(j * 16, 16)] = jnp.zeros((16,), jnp.float32)
                return carry2
            return lax.fori_loop(0, d // 16, zc, carry)
        lax.fori_loop(0, K, zr, 0)

        def zd(i, carry):
            deg_v[i, :] = jnp.zeros((16,), jnp.float32)
            ones_v[i, :] = jnp.ones((16,), jnp.float32)
            return carry
        lax.fori_loop(0, K, zd, 0)

        for off, blk in plan:
            pltpu.sync_copy(rows_v.at[pl.ds(0, blk)],
                            agg_sh.at[pl.ds(base + off, blk)])
            pltpu.sync_copy(deg_v.at[pl.ds(0, blk)],
                            deg_sh.at[pl.ds(base + off, blk)])
        plsc.subcore_barrier()

        pltpu.sync_copy(src_hbm.at[wid], src_v)
        pltpu.sync_copy(dst_hbm.at[wid], dst_v)

        def body(gi, carry):
            pltpu.async_copy(x_hbm.at[src_v.at[gi]], rows_v, sem).wait()
            pltpu.sync_copy(rows_v, agg_sh.at[dst_v.at[gi]], add=True)
            pltpu.sync_copy(ones_v, deg_sh.at[dst_v.at[gi]], add=True)
            return carry
        lax.fori_loop(0, g, body, 0)
        plsc.subcore_barrier()

        for off, blk in plan:
            pltpu.sync_copy(agg_sh.at[pl.ds(base + off, blk)],
                            rows_v.at[pl.ds(0, blk)])
            pltpu.sync_copy(rows_v.at[pl.ds(0, blk)],
                            agg_out.at[c, pl.ds(base + off, blk)])
            pltpu.sync_copy(deg_sh.at[pl.ds(base + off, blk)],
                            deg_v.at[pl.ds(0, blk)])
            pltpu.sync_copy(deg_v.at[pl.ds(0, blk)],
                            deg_out.at[c, pl.ds(base + off, blk)])

    return sc_agg


def _tc_finish(agg_parts, deg_parts, x, wl_t, wr_t, b2, rblock):
    n, d = x.shape

    def body(agg_ref, deg_ref, x_ref, wl_ref, wr_ref, b_ref, o_ref):
        a = agg_ref[0] + agg_ref[1]
        dg = jnp.maximum(deg_ref[0, :, 0:1] + deg_ref[1, :, 0:1], 1.0)
        mean = a / dg
        acc = jnp.dot(mean, wl_ref[...], preferred_element_type=jnp.float32)
        acc = acc + jnp.dot(x_ref[...], wr_ref[...],
                            preferred_element_type=jnp.float32)
        o_ref[...] = acc + b_ref[...]

    return pl.pallas_call(
        body,
        grid=(n // rblock,),
        in_specs=[
            pl.BlockSpec((NC, rblock, d), lambda i: (0, i, 0)),
            pl.BlockSpec((NC, rblock, 16), lambda i: (0, i, 0)),
            pl.BlockSpec((rblock, d), lambda i: (i, 0)),
            pl.BlockSpec((d, d), lambda i: (0, 0)),
            pl.BlockSpec((d, d), lambda i: (0, 0)),
            pl.BlockSpec((1, d), lambda i: (0, 0)),
        ],
        out_specs=pl.BlockSpec((rblock, d), lambda i: (i, 0)),
        out_shape=jax.ShapeDtypeStruct((n, d), jnp.float32),
    )(agg_parts, deg_parts, x, wl_t, wr_t, b2)


def kernel(x, edge_index, W_l, b_l, W_r):
    n, d = x.shape
    e = edge_index.shape[1]

    g = -(-e // (NW * K))
    e_pad = NW * g * K
    npad = -(-(n + 1) // NS) * NS

    src = edge_index[0]
    dst = edge_index[1]
    pad = e_pad - e
    if pad:
        src = jnp.concatenate([src, jnp.zeros((pad,), jnp.int32)])
        dst = jnp.concatenate([dst, jnp.full((pad,), n, jnp.int32)])
    src3d = src.reshape(NW, g, K)
    dst3d = dst.reshape(NW, g, K)

    sc_agg = _build_sc_kernel(n, d, g, npad)
    agg_parts, deg_parts = sc_agg(x, src3d, dst3d)

    rblock = 400 if n % 400 == 0 else 8
    return _tc_finish(agg_parts, deg_parts, x, W_l.T, W_r.T,
                      b_l.reshape(1, d), rblock)
